# speedup vs baseline: 1.5523x; 1.5523x over previous
import functools

import jax
import jax.numpy as jnp
from jax import lax
from jax.experimental import pallas as pl
from jax.experimental.pallas import tpu as pltpu
from jax.experimental.pallas import tpu_sc as plsc

_NC = 2
_NS = 16
_NW = _NC * _NS

_B, _L, _D = 4, 4096, 768
_N = _B * _L
_PER_W = _N // _NW
_CHUNK = 64
_NCHUNK = _PER_W // _CHUNK


@functools.partial(
    pl.kernel,
    mesh=plsc.VectorSubcoreMesh(core_axis_name="c", subcore_axis_name="s"),
    out_type=jax.ShapeDtypeStruct((_N, _D), jnp.float32),
    scratch_types=[
        pltpu.VMEM((_NCHUNK, _CHUNK), jnp.int32),
        pltpu.VMEM((_CHUNK, _D), jnp.float32),
        pltpu.VMEM((_CHUNK, _D), jnp.float32),
        pltpu.SemaphoreType.DMA,
        pltpu.SemaphoreType.DMA,
    ],
)
def _sc_gather(table_hbm, idx_hbm, out_hbm, idx_v, rows0, rows1, sem0, sem1):
    wid = lax.axis_index("s") * _NC + lax.axis_index("c")
    base = wid * _PER_W
    pltpu.sync_copy(idx_hbm.at[wid], idx_v)

    rows = (rows0, rows1)
    sems = (sem0, sem1)
    copies = [None, None]
    copies[0] = pltpu.async_copy(table_hbm.at[idx_v.at[0]], rows0, sem0)
    for c in range(_NCHUNK):
        cur = c % 2
        copies[cur].wait()
        if c + 1 < _NCHUNK:
            nxt = (c + 1) % 2
            copies[nxt] = pltpu.async_copy(
                table_hbm.at[idx_v.at[c + 1]], rows[nxt], sems[nxt])
        pltpu.sync_copy(rows[cur], out_hbm.at[pl.ds(base + c * _CHUNK, _CHUNK)])


def kernel(src, embedding_table):
    idx = src.reshape(_NW, _NCHUNK, _CHUNK).astype(jnp.int32)
    out = _sc_gather(embedding_table, idx)
    return out.reshape(_B, _L, _D)
